# baseline (speedup 1.0000x reference)
import functools

import jax
import jax.numpy as jnp
from jax import lax
from jax.experimental import pallas as pl
from jax.experimental.pallas import tpu as pltpu
from jax.experimental.pallas import tpu_sc as plsc

N = 10000
E = 320000
DF = 128
DE = 16

NC = 2
NS = 16
NW = NC * NS
K = 40
NBUF = 2
NP = 10240
RPS = NP // NS

BLK = 3200
NBLK = E // BLK

NE0 = 166400
NE1 = E - NE0
NBLK0 = NE0 // BLK

_DN0 = (((0,), (0,)), ((), ()))


def _estats_body(efT_ref, eW1_ref, eb1c_ref, out_ref):
    e1t = lax.dot_general(eW1_ref[...], efT_ref[...], _DN0,
                          preferred_element_type=jnp.float32) + eb1c_ref[...]

    @pl.when(pl.program_id(0) == 0)
    def _():
        out_ref[...] = jnp.zeros_like(out_ref)

    out_ref[:, 0:1] += jnp.sum(e1t, axis=1, keepdims=True)
    out_ref[:, 1:2] += jnp.sum(e1t * e1t, axis=1, keepdims=True)


def _emlp_body(efT_ref, stats_ref, eW1_ref, eb1c_ref, eg1c_ref, ebt1c_ref,
               eW2_ref, eb2_ref, out_ref):
    m = stats_ref[:, 0:1] / E
    var = stats_ref[:, 1:2] / E - m * m
    scale = eg1c_ref[...] * lax.rsqrt(var + 1e-5)
    shift = ebt1c_ref[...] - m * scale
    e1t = lax.dot_general(eW1_ref[...], efT_ref[...], _DN0,
                          preferred_element_type=jnp.float32) + eb1c_ref[...]
    x = jnp.maximum(e1t * scale + shift, 0.0)
    out_ref[...] = lax.dot_general(x, eW2_ref[...], _DN0,
                                   preferred_element_type=jnp.float32) + eb2_ref[...]


def _sc_body(eph, src_hbm, dst_hbm, e_hbm, init_hbm, node_hbm, out_hbm,
             *scr):
    nchunk = eph // K
    c = lax.axis_index("c")
    s = lax.axis_index("s")
    wid = s * NC + c
    srca, dsta = scr[0], scr[1]
    nrows = scr[2:2 + NBUF]
    erows = scr[2 + NBUF:2 + 2 * NBUF]
    mrows = scr[2 + 2 * NBUF:2 + 3 * NBUF]
    acc = scr[2 + 3 * NBUF]
    sems = scr[3 + 3 * NBUF:]
    gsem = sems[0:NBUF]
    esem = sems[NBUF:2 * NBUF]
    ssem = sems[2 * NBUF:3 * NBUF]

    pltpu.sync_copy(init_hbm.at[c, pl.ds(s * RPS, RPS)],
                    acc.at[pl.ds(s * RPS, RPS)])
    pltpu.sync_copy(src_hbm.at[pl.ds(wid * eph, eph)], srca)
    pltpu.sync_copy(dst_hbm.at[pl.ds(wid * eph, eph)], dsta)
    plsc.subcore_barrier()

    def start_loads(ci, b):
        pltpu.async_copy(node_hbm.at[srca.at[pl.ds(ci * K, K)]],
                         nrows[b], gsem[b])
        pltpu.async_copy(e_hbm.at[pl.ds(wid * eph + ci * K, K)],
                         erows[b], esem[b])

    for b in range(NBUF):
        start_loads(b, b)

    def pair(g, carry):
        for b in range(NBUF):
            ci = g * NBUF + b
            pltpu.make_async_copy(node_hbm.at[srca.at[pl.ds(ci * K, K)]],
                                  nrows[b], gsem[b]).wait()
            pltpu.make_async_copy(e_hbm.at[pl.ds(wid * eph + ci * K, K)],
                                  erows[b], esem[b]).wait()

            @pl.when(g > 0)
            def _():
                pltpu.make_async_copy(mrows[b],
                                      acc.at[dsta.at[pl.ds(ci * K, K)]],
                                      ssem[b]).wait()

            @plsc.parallel_loop(0, K, unroll=8)
            def _row(i):
                for cc in range(DF // 16):
                    sl = pl.ds(cc * 16, 16)
                    mrows[b][i, sl] = jnp.maximum(
                        nrows[b][i, sl] + erows[b][i, sl], 0.0)
            pltpu.async_copy(mrows[b], acc.at[dsta.at[pl.ds(ci * K, K)]],
                             ssem[b], add=True)

            @pl.when(ci + NBUF < nchunk)
            def _():
                start_loads(ci + NBUF, b)
        return carry

    lax.fori_loop(0, nchunk // NBUF, pair, 0)
    for b in range(NBUF):
        pltpu.make_async_copy(mrows[b], acc.at[dsta.at[pl.ds(0, K)]],
                              ssem[b]).wait()
    plsc.subcore_barrier()
    pltpu.sync_copy(acc.at[pl.ds(s * RPS, RPS)],
                    out_hbm.at[c, pl.ds(s * RPS, RPS)])


def _nmlp_body(x_ref, p_ref, nW1_ref, nb1_ref, ng1_ref, nbt1_ref,
               nW2_ref, nb2_ref, eps_ref, out_ref):
    h = ((1.0 + eps_ref[0, 0]) * x_ref[...]
         + p_ref[0, :N, :] + p_ref[1, :N, :])
    z = jnp.dot(h, nW1_ref[...], preferred_element_type=jnp.float32) + nb1_ref[...]
    m = jnp.mean(z, axis=0, keepdims=True)
    v = jnp.mean((z - m) * (z - m), axis=0, keepdims=True)
    zn = ng1_ref[...] * (z - m) * lax.rsqrt(v + 1e-5) + nbt1_ref[...]
    zn = jnp.maximum(zn, 0.0)
    out_ref[...] = jnp.dot(zn, nW2_ref[...],
                           preferred_element_type=jnp.float32) + nb2_ref[...]


def kernel(node_feat, edge_feat, edge_index, eW1, eb1, eg1, ebt1, eW2, eb2,
           nW1, nb1, ng1, nbt1, nW2, nb2, eps):
    f32 = jnp.float32
    efT = edge_feat.T
    eb1c = eb1.reshape(-1, 1)
    eg1c = eg1.reshape(-1, 1)
    ebt1c = ebt1.reshape(-1, 1)
    eb2r = eb2.reshape(1, -1)
    nb1r = nb1.reshape(1, -1)
    ng1r = ng1.reshape(1, -1)
    nbt1r = nbt1.reshape(1, -1)
    nb2r = nb2.reshape(1, -1)
    epsr = eps.reshape(1, 1)

    whole = pl.BlockSpec(index_map=lambda i: (0, 0))
    stats = pl.pallas_call(
        _estats_body,
        grid=(NBLK,),
        in_specs=[
            pl.BlockSpec((DE, BLK), lambda i: (0, i)),
            whole, whole,
        ],
        out_specs=pl.BlockSpec((2 * DE, 8), lambda i: (0, 0)),
        out_shape=jax.ShapeDtypeStruct((2 * DE, 8), f32),
    )(efT, eW1, eb1c)

    def emlp(nblk, blk0):
        return pl.pallas_call(
            _emlp_body,
            grid=(nblk,),
            in_specs=[
                pl.BlockSpec((DE, BLK), lambda i: (0, i + blk0)),
                pl.BlockSpec((2 * DE, 8), lambda i: (0, 0)),
                whole, whole, whole, whole, whole, whole,
            ],
            out_specs=pl.BlockSpec((BLK, DF), lambda i: (i, 0)),
            out_shape=jax.ShapeDtypeStruct((nblk * BLK, DF), f32),
        )(efT, stats, eW1, eb1c, eg1c, ebt1c, eW2, eb2r)

    e0 = emlp(NBLK0, 0)

    src = edge_index[0]
    dst = edge_index[1]
    zeros = jnp.zeros((NC, NP, DF), f32)

    mesh = plsc.VectorSubcoreMesh(core_axis_name="c", subcore_axis_name="s")

    def sc_call(eph, src_h, dst_h, e_h, init_h):
        return pl.kernel(
            functools.partial(_sc_body, eph),
            out_type=jax.ShapeDtypeStruct((NC, NP, DF), f32),
            mesh=mesh,
            scratch_types=(
                [pltpu.VMEM((eph,), jnp.int32)] * 2
                + [pltpu.VMEM((K, DF), f32)] * (3 * NBUF)
                + [pltpu.VMEM_SHARED((NP, DF), f32)]
                + [pltpu.SemaphoreType.DMA] * (3 * NBUF)
            ),
        )(src_h, dst_h, e_h, init_h, node_feat)

    parts0 = sc_call(NE0 // NW, src[:NE0], dst[:NE0], e0, zeros)
    e1 = emlp(NBLK - NBLK0, NBLK0)
    parts = sc_call(NE1 // NW, src[NE0:], dst[NE0:], e1, parts0)

    out = pl.pallas_call(
        _nmlp_body,
        out_shape=jax.ShapeDtypeStruct((N, DF), f32),
    )(node_feat, parts, nW1, nb1r, ng1r, nbt1r, nW2, nb2r, epsr)
    return out


# speedup vs baseline: 1.0436x; 1.0436x over previous
import functools

import jax
import jax.numpy as jnp
from jax import lax
from jax.experimental import pallas as pl
from jax.experimental.pallas import tpu as pltpu
from jax.experimental.pallas import tpu_sc as plsc

N = 10000
E = 320000
DF = 128
DE = 16

NC = 2
NS = 16
NW = NC * NS
K = 40
NBUF = 2
NP = 10240
RPS = NP // NS

BLK = 3200
NBLK = E // BLK

PHASES = (38400, 102400, 179200)

_DN0 = (((0,), (0,)), ((), ()))


def _estats_body(efT_ref, eW1_ref, eb1c_ref, out_ref):
    e1t = lax.dot_general(eW1_ref[...], efT_ref[...], _DN0,
                          preferred_element_type=jnp.float32) + eb1c_ref[...]

    @pl.when(pl.program_id(0) == 0)
    def _():
        out_ref[...] = jnp.zeros_like(out_ref)

    out_ref[:, 0:1] += jnp.sum(e1t, axis=1, keepdims=True)
    out_ref[:, 1:2] += jnp.sum(e1t * e1t, axis=1, keepdims=True)


def _emlp_body(efT_ref, stats_ref, eW1_ref, eb1c_ref, eg1c_ref, ebt1c_ref,
               eW2_ref, eb2_ref, out_ref):
    m = stats_ref[:, 0:1] / E
    var = stats_ref[:, 1:2] / E - m * m
    scale = eg1c_ref[...] * lax.rsqrt(var + 1e-5)
    shift = ebt1c_ref[...] - m * scale
    e1t = lax.dot_general(eW1_ref[...], efT_ref[...], _DN0,
                          preferred_element_type=jnp.float32) + eb1c_ref[...]
    x = jnp.maximum(e1t * scale + shift, 0.0)
    out_ref[...] = lax.dot_general(x, eW2_ref[...], _DN0,
                                   preferred_element_type=jnp.float32) + eb2_ref[...]


def _sc_body(eph, src_hbm, dst_hbm, e_hbm, init_hbm, node_hbm, out_hbm,
             *scr):
    nchunk = eph // K
    c = lax.axis_index("c")
    s = lax.axis_index("s")
    wid = s * NC + c
    srca, dsta = scr[0], scr[1]
    nrows = scr[2:2 + NBUF]
    erows = scr[2 + NBUF:2 + 2 * NBUF]
    mrows = scr[2 + 2 * NBUF:2 + 3 * NBUF]
    acc = scr[2 + 3 * NBUF]
    sems = scr[3 + 3 * NBUF:]
    gsem = sems[0:NBUF]
    esem = sems[NBUF:2 * NBUF]
    ssem = sems[2 * NBUF:3 * NBUF]

    pltpu.sync_copy(init_hbm.at[c, pl.ds(s * RPS, RPS)],
                    acc.at[pl.ds(s * RPS, RPS)])
    pltpu.sync_copy(src_hbm.at[pl.ds(wid * eph, eph)], srca)
    pltpu.sync_copy(dst_hbm.at[pl.ds(wid * eph, eph)], dsta)
    plsc.subcore_barrier()

    def start_loads(ci, b):
        pltpu.async_copy(node_hbm.at[srca.at[pl.ds(ci * K, K)]],
                         nrows[b], gsem[b])
        pltpu.async_copy(e_hbm.at[pl.ds(wid * eph + ci * K, K)],
                         erows[b], esem[b])

    for b in range(NBUF):
        start_loads(b, b)

    def pair(g, carry):
        for b in range(NBUF):
            ci = g * NBUF + b
            pltpu.make_async_copy(node_hbm.at[srca.at[pl.ds(ci * K, K)]],
                                  nrows[b], gsem[b]).wait()
            pltpu.make_async_copy(e_hbm.at[pl.ds(wid * eph + ci * K, K)],
                                  erows[b], esem[b]).wait()

            @pl.when(g > 0)
            def _():
                pltpu.make_async_copy(mrows[b],
                                      acc.at[dsta.at[pl.ds(ci * K, K)]],
                                      ssem[b]).wait()

            @plsc.parallel_loop(0, K, unroll=4)
            def _row(i):
                for cc in range(DF // 16):
                    sl = pl.ds(cc * 16, 16)
                    mrows[b][i, sl] = jnp.maximum(
                        nrows[b][i, sl] + erows[b][i, sl], 0.0)
            pltpu.async_copy(mrows[b], acc.at[dsta.at[pl.ds(ci * K, K)]],
                             ssem[b], add=True)

            @pl.when(ci + NBUF < nchunk)
            def _():
                start_loads(ci + NBUF, b)
        return carry

    lax.fori_loop(0, nchunk // NBUF, pair, 0)
    for b in range(NBUF):
        pltpu.make_async_copy(mrows[b], acc.at[dsta.at[pl.ds(0, K)]],
                              ssem[b]).wait()
    plsc.subcore_barrier()
    pltpu.sync_copy(acc.at[pl.ds(s * RPS, RPS)],
                    out_hbm.at[c, pl.ds(s * RPS, RPS)])


def _nmlp_body(x_ref, p_ref, nW1_ref, nb1_ref, ng1_ref, nbt1_ref,
               nW2_ref, nb2_ref, eps_ref, out_ref):
    h = ((1.0 + eps_ref[0, 0]) * x_ref[...]
         + p_ref[0, :N, :] + p_ref[1, :N, :])
    z = jnp.dot(h, nW1_ref[...], preferred_element_type=jnp.float32) + nb1_ref[...]
    m = jnp.mean(z, axis=0, keepdims=True)
    v = jnp.mean((z - m) * (z - m), axis=0, keepdims=True)
    zn = ng1_ref[...] * (z - m) * lax.rsqrt(v + 1e-5) + nbt1_ref[...]
    zn = jnp.maximum(zn, 0.0)
    out_ref[...] = jnp.dot(zn, nW2_ref[...],
                           preferred_element_type=jnp.float32) + nb2_ref[...]


def kernel(node_feat, edge_feat, edge_index, eW1, eb1, eg1, ebt1, eW2, eb2,
           nW1, nb1, ng1, nbt1, nW2, nb2, eps):
    f32 = jnp.float32
    efT = edge_feat.T
    eb1c = eb1.reshape(-1, 1)
    eg1c = eg1.reshape(-1, 1)
    ebt1c = ebt1.reshape(-1, 1)
    eb2r = eb2.reshape(1, -1)
    nb1r = nb1.reshape(1, -1)
    ng1r = ng1.reshape(1, -1)
    nbt1r = nbt1.reshape(1, -1)
    nb2r = nb2.reshape(1, -1)
    epsr = eps.reshape(1, 1)

    whole = pl.BlockSpec(index_map=lambda i: (0, 0))
    stats = pl.pallas_call(
        _estats_body,
        grid=(NBLK,),
        in_specs=[
            pl.BlockSpec((DE, BLK), lambda i: (0, i)),
            whole, whole,
        ],
        out_specs=pl.BlockSpec((2 * DE, 8), lambda i: (0, 0)),
        out_shape=jax.ShapeDtypeStruct((2 * DE, 8), f32),
    )(efT, eW1, eb1c)

    def emlp(nblk, blk0):
        return pl.pallas_call(
            _emlp_body,
            grid=(nblk,),
            in_specs=[
                pl.BlockSpec((DE, BLK), lambda i: (0, i + blk0)),
                pl.BlockSpec((2 * DE, 8), lambda i: (0, 0)),
                whole, whole, whole, whole, whole, whole,
            ],
            out_specs=pl.BlockSpec((BLK, DF), lambda i: (i, 0)),
            out_shape=jax.ShapeDtypeStruct((nblk * BLK, DF), f32),
        )(efT, stats, eW1, eb1c, eg1c, ebt1c, eW2, eb2r)

    src = edge_index[0]
    dst = edge_index[1]
    zeros = jnp.zeros((NC, NP, DF), f32)

    mesh = plsc.VectorSubcoreMesh(core_axis_name="c", subcore_axis_name="s")

    def sc_call(eph, src_h, dst_h, e_h, init_h):
        return pl.kernel(
            functools.partial(_sc_body, eph),
            out_type=jax.ShapeDtypeStruct((NC, NP, DF), f32),
            mesh=mesh,
            scratch_types=(
                [pltpu.VMEM((eph,), jnp.int32)] * 2
                + [pltpu.VMEM((K, DF), f32)] * (3 * NBUF)
                + [pltpu.VMEM_SHARED((NP, DF), f32)]
                + [pltpu.SemaphoreType.DMA] * (3 * NBUF)
            ),
        )(src_h, dst_h, e_h, init_h, node_feat)

    parts = zeros
    eo = 0
    blk0 = 0
    for ne in PHASES:
        nblk_h = ne // BLK
        e_h = emlp(nblk_h, blk0)
        parts = sc_call(ne // NW, src[eo:eo + ne], dst[eo:eo + ne], e_h, parts)
        eo += ne
        blk0 += nblk_h

    out = pl.pallas_call(
        _nmlp_body,
        out_shape=jax.ShapeDtypeStruct((N, DF), f32),
    )(node_feat, parts, nW1, nb1r, ng1r, nbt1r, nW2, nb2r, epsr)
    return out


# speedup vs baseline: 1.0452x; 1.0016x over previous
import functools

import jax
import jax.numpy as jnp
from jax import lax
from jax.experimental import pallas as pl
from jax.experimental.pallas import tpu as pltpu
from jax.experimental.pallas import tpu_sc as plsc

N = 10000
E = 320000
DF = 128
DE = 16

NC = 2
NS = 16
NW = NC * NS
K = 40
NBUF = 2
NP = 10240
RPS = NP // NS

BLK = 3200
NBLK = E // BLK

PHASES = (102400, 217600)

_DN0 = (((0,), (0,)), ((), ()))


def _estats_body(efT_ref, eW1_ref, eb1c_ref, out_ref):
    e1t = lax.dot_general(eW1_ref[...], efT_ref[...], _DN0,
                          preferred_element_type=jnp.float32) + eb1c_ref[...]

    @pl.when(pl.program_id(0) == 0)
    def _():
        out_ref[...] = jnp.zeros_like(out_ref)

    out_ref[:, 0:1] += jnp.sum(e1t, axis=1, keepdims=True)
    out_ref[:, 1:2] += jnp.sum(e1t * e1t, axis=1, keepdims=True)


def _emlp_body(efT_ref, stats_ref, eW1_ref, eb1c_ref, eg1c_ref, ebt1c_ref,
               eW2_ref, eb2_ref, out_ref):
    m = stats_ref[:, 0:1] / E
    var = stats_ref[:, 1:2] / E - m * m
    scale = eg1c_ref[...] * lax.rsqrt(var + 1e-5)
    shift = ebt1c_ref[...] - m * scale
    e1t = lax.dot_general(eW1_ref[...], efT_ref[...], _DN0,
                          preferred_element_type=jnp.float32) + eb1c_ref[...]
    x = jnp.maximum(e1t * scale + shift, 0.0)
    out_ref[...] = lax.dot_general(x, eW2_ref[...], _DN0,
                                   preferred_element_type=jnp.float32) + eb2_ref[...]


def _sc_body(eph, src_hbm, dst_hbm, e_hbm, init_hbm, node_hbm, out_hbm,
             *scr):
    nchunk = eph // K
    c = lax.axis_index("c")
    s = lax.axis_index("s")
    wid = s * NC + c
    srca, dsta = scr[0], scr[1]
    nrows = scr[2:2 + NBUF]
    erows = scr[2 + NBUF:2 + 2 * NBUF]
    mrows = scr[2 + 2 * NBUF:2 + 3 * NBUF]
    acc = scr[2 + 3 * NBUF]
    sems = scr[3 + 3 * NBUF:]
    gsem = sems[0:NBUF]
    esem = sems[NBUF:2 * NBUF]
    ssem = sems[2 * NBUF:3 * NBUF]

    pltpu.sync_copy(init_hbm.at[c, pl.ds(s * RPS, RPS)],
                    acc.at[pl.ds(s * RPS, RPS)])
    pltpu.sync_copy(src_hbm.at[pl.ds(wid * eph, eph)], srca)
    pltpu.sync_copy(dst_hbm.at[pl.ds(wid * eph, eph)], dsta)
    plsc.subcore_barrier()

    def start_loads(ci, b):
        pltpu.async_copy(node_hbm.at[srca.at[pl.ds(ci * K, K)]],
                         nrows[b], gsem[b])
        pltpu.async_copy(e_hbm.at[pl.ds(wid * eph + ci * K, K)],
                         erows[b], esem[b])

    for b in range(NBUF):
        start_loads(b, b)

    def pair(g, carry):
        for b in range(NBUF):
            ci = g * NBUF + b
            pltpu.make_async_copy(node_hbm.at[srca.at[pl.ds(ci * K, K)]],
                                  nrows[b], gsem[b]).wait()
            pltpu.make_async_copy(e_hbm.at[pl.ds(wid * eph + ci * K, K)],
                                  erows[b], esem[b]).wait()

            @pl.when(g > 0)
            def _():
                pltpu.make_async_copy(mrows[b],
                                      acc.at[dsta.at[pl.ds(ci * K, K)]],
                                      ssem[b]).wait()

            @plsc.parallel_loop(0, K, unroll=4)
            def _row(i):
                for cc in range(DF // 16):
                    sl = pl.ds(cc * 16, 16)
                    mrows[b][i, sl] = jnp.maximum(
                        nrows[b][i, sl] + erows[b][i, sl], 0.0)
            pltpu.async_copy(mrows[b], acc.at[dsta.at[pl.ds(ci * K, K)]],
                             ssem[b], add=True)

            @pl.when(ci + NBUF < nchunk)
            def _():
                start_loads(ci + NBUF, b)
        return carry

    lax.fori_loop(0, nchunk // NBUF, pair, 0)
    for b in range(NBUF):
        pltpu.make_async_copy(mrows[b], acc.at[dsta.at[pl.ds(0, K)]],
                              ssem[b]).wait()
    plsc.subcore_barrier()
    pltpu.sync_copy(acc.at[pl.ds(s * RPS, RPS)],
                    out_hbm.at[c, pl.ds(s * RPS, RPS)])


def _nmlp_body(x_ref, p_ref, nW1_ref, nb1_ref, ng1_ref, nbt1_ref,
               nW2_ref, nb2_ref, eps_ref, out_ref):
    h = ((1.0 + eps_ref[0, 0]) * x_ref[...]
         + p_ref[0, :N, :] + p_ref[1, :N, :])
    z = jnp.dot(h, nW1_ref[...], preferred_element_type=jnp.float32) + nb1_ref[...]
    m = jnp.mean(z, axis=0, keepdims=True)
    v = jnp.mean((z - m) * (z - m), axis=0, keepdims=True)
    zn = ng1_ref[...] * (z - m) * lax.rsqrt(v + 1e-5) + nbt1_ref[...]
    zn = jnp.maximum(zn, 0.0)
    out_ref[...] = jnp.dot(zn, nW2_ref[...],
                           preferred_element_type=jnp.float32) + nb2_ref[...]


def kernel(node_feat, edge_feat, edge_index, eW1, eb1, eg1, ebt1, eW2, eb2,
           nW1, nb1, ng1, nbt1, nW2, nb2, eps):
    f32 = jnp.float32
    efT = edge_feat.T
    eb1c = eb1.reshape(-1, 1)
    eg1c = eg1.reshape(-1, 1)
    ebt1c = ebt1.reshape(-1, 1)
    eb2r = eb2.reshape(1, -1)
    nb1r = nb1.reshape(1, -1)
    ng1r = ng1.reshape(1, -1)
    nbt1r = nbt1.reshape(1, -1)
    nb2r = nb2.reshape(1, -1)
    epsr = eps.reshape(1, 1)

    whole = pl.BlockSpec(index_map=lambda i: (0, 0))
    stats = pl.pallas_call(
        _estats_body,
        grid=(NBLK,),
        in_specs=[
            pl.BlockSpec((DE, BLK), lambda i: (0, i)),
            whole, whole,
        ],
        out_specs=pl.BlockSpec((2 * DE, 8), lambda i: (0, 0)),
        out_shape=jax.ShapeDtypeStruct((2 * DE, 8), f32),
    )(efT, eW1, eb1c)

    def emlp(nblk, blk0):
        return pl.pallas_call(
            _emlp_body,
            grid=(nblk,),
            in_specs=[
                pl.BlockSpec((DE, BLK), lambda i: (0, i + blk0)),
                pl.BlockSpec((2 * DE, 8), lambda i: (0, 0)),
                whole, whole, whole, whole, whole, whole,
            ],
            out_specs=pl.BlockSpec((BLK, DF), lambda i: (i, 0)),
            out_shape=jax.ShapeDtypeStruct((nblk * BLK, DF), f32),
        )(efT, stats, eW1, eb1c, eg1c, ebt1c, eW2, eb2r)

    src = edge_index[0]
    dst = edge_index[1]
    zeros = jnp.zeros((NC, NP, DF), f32)

    mesh = plsc.VectorSubcoreMesh(core_axis_name="c", subcore_axis_name="s")

    def sc_call(eph, src_h, dst_h, e_h, init_h):
        return pl.kernel(
            functools.partial(_sc_body, eph),
            out_type=jax.ShapeDtypeStruct((NC, NP, DF), f32),
            mesh=mesh,
            scratch_types=(
                [pltpu.VMEM((eph,), jnp.int32)] * 2
                + [pltpu.VMEM((K, DF), f32)] * (3 * NBUF)
                + [pltpu.VMEM_SHARED((NP, DF), f32)]
                + [pltpu.SemaphoreType.DMA] * (3 * NBUF)
            ),
        )(src_h, dst_h, e_h, init_h, node_feat)

    parts = zeros
    eo = 0
    blk0 = 0
    for ne in PHASES:
        nblk_h = ne // BLK
        e_h = emlp(nblk_h, blk0)
        parts = sc_call(ne // NW, src[eo:eo + ne], dst[eo:eo + ne], e_h, parts)
        eo += ne
        blk0 += nblk_h

    out = pl.pallas_call(
        _nmlp_body,
        out_shape=jax.ShapeDtypeStruct((N, DF), f32),
    )(node_feat, parts, nW1, nb1r, ng1r, nbt1r, nW2, nb2r, epsr)
    return out
